# baseline (speedup 1.0000x reference)
import functools

import jax
import jax.numpy as jnp
from jax import lax
from jax.experimental import pallas as pl
from jax.experimental.pallas import tpu as pltpu
from jax.experimental.pallas import tpu_sc as plsc

EMB = 64


def _gather_kernel(batch, num_workers):
    b_per_w = batch // num_workers
    mesh = plsc.VectorSubcoreMesh(core_axis_name="c", subcore_axis_name="s")
    num_cores = mesh.num_cores

    @functools.partial(
        pl.kernel,
        out_type=(
            jax.ShapeDtypeStruct((batch, EMB), jnp.float32),
            jax.ShapeDtypeStruct((batch,), jnp.float32),
        ),
        mesh=mesh,
        compiler_params=pltpu.CompilerParams(use_tc_tiling_on_sc=False),
        scratch_types=[
            pltpu.VMEM((b_per_w,), jnp.int32),
            pltpu.VMEM((b_per_w, EMB), jnp.float32),
            pltpu.VMEM((b_per_w,), jnp.float32),
            pltpu.SemaphoreType.DMA,
            pltpu.SemaphoreType.DMA,
        ],
    )
    def k(ids, table, bias, rows_out, bias_out,
          idx_v, rows_v, b_v, sem_r, sem_b):
        wid = lax.axis_index("s") * num_cores + lax.axis_index("c")
        base = wid * b_per_w
        pltpu.sync_copy(ids.at[pl.ds(base, b_per_w)], idx_v)
        cp_r = pltpu.async_copy(table.at[idx_v], rows_v, sem_r)
        cp_b = pltpu.async_copy(bias.at[idx_v], b_v, sem_b)
        cp_r.wait()
        pltpu.sync_copy(rows_v, rows_out.at[pl.ds(base, b_per_w)])
        cp_b.wait()
        pltpu.sync_copy(b_v, bias_out.at[pl.ds(base, b_per_w)])

    return k


def _pred_block(u_ref, i_ref, ub_ref, ib_ref, mean_ref, o_ref):
    prod = u_ref[...] * i_ref[...]
    o_ref[...] = (jnp.sum(prod, axis=1) + ub_ref[...] + ib_ref[...]
                  + mean_ref[0])


def _pred_kernel(batch):
    blk = 2048
    grid = batch // blk
    return pl.pallas_call(
        _pred_block,
        grid=(grid,),
        in_specs=[
            pl.BlockSpec((blk, EMB), lambda i: (i, 0)),
            pl.BlockSpec((blk, EMB), lambda i: (i, 0)),
            pl.BlockSpec((blk,), lambda i: (i,)),
            pl.BlockSpec((blk,), lambda i: (i,)),
            pl.BlockSpec(memory_space=pltpu.SMEM),
        ],
        out_specs=pl.BlockSpec((blk,), lambda i: (i,)),
        out_shape=jax.ShapeDtypeStruct((batch,), jnp.float32),
    )


def kernel(u_id, i_id, user_emb, user_bias, item_emb, item_bias, mean):
    batch = u_id.shape[0]
    info = plsc.get_sparse_core_info()
    num_workers = info.num_cores * info.num_subcores
    gk = _gather_kernel(batch, num_workers)
    U, U_b = gk(u_id.astype(jnp.int32), user_emb,
                jnp.reshape(user_bias, (-1,)))
    I, I_b = gk(i_id.astype(jnp.int32), item_emb,
                jnp.reshape(item_bias, (-1,)))
    pred = _pred_kernel(batch)(U, I, U_b, I_b, mean)
    return (pred, U, I)
